# baseline (speedup 1.0000x reference)
import functools

import jax
import jax.numpy as jnp
from jax import lax
from jax.experimental import pallas as pl
from jax.experimental.pallas import tpu as pltpu
from jax.experimental.pallas import tpu_sc as plsc

B, S, C, D = 32, 50, 20, 128
PAIRS = B * S
NW = 32
PPW = PAIRS // NW
CH = 2
NCHUNK = PPW // CH
LANES = 16


def _sc_embed_pool(idx0, idx1, idx2, idx3, t0, t1, t2, t3):
    mesh = plsc.VectorSubcoreMesh(core_axis_name="c", subcore_axis_name="s")

    @functools.partial(
        pl.kernel,
        out_type=jax.ShapeDtypeStruct((PAIRS * D,), jnp.float32),
        mesh=mesh,
        scratch_types=[
            pltpu.VMEM((PPW * C,), jnp.int32),
            pltpu.VMEM((PPW * C,), jnp.int32),
            pltpu.VMEM((PPW * C,), jnp.int32),
            pltpu.VMEM((PPW * C,), jnp.int32),
            pltpu.VMEM((CH * C, D), jnp.float32),
            pltpu.VMEM((CH * C, D), jnp.float32),
            pltpu.VMEM((CH * C, D), jnp.float32),
            pltpu.VMEM((CH * C, D), jnp.float32),
            pltpu.VMEM((PPW * D,), jnp.float32),
            pltpu.SemaphoreType.DMA,
        ],
    )
    def body(i0h, i1h, i2h, i3h, e0, e1, e2, e3, out,
             i0, i1, i2, i3, r0, r1, r2, r3, outv, sem):
        wid = lax.axis_index("s") * 2 + lax.axis_index("c")
        ibase = wid * (PPW * C)
        irefs = (i0, i1, i2, i3)
        rrefs = (r0, r1, r2, r3)
        tabs = (e0, e1, e2, e3)
        for iref, ihbm in zip(irefs, (i0h, i1h, i2h, i3h)):
            pltpu.sync_copy(ihbm.at[pl.ds(ibase, PPW * C)], iref)

        def chunk_body(g, carry):
            cps = []
            for iref, tab, rref in zip(irefs, tabs, rrefs):
                cps.append(
                    pltpu.async_copy(
                        tab.at[iref.at[pl.ds(g * (CH * C), CH * C)]], rref, sem
                    )
                )
            for cp in cps:
                cp.wait()
            for lp in range(CH):
                pair = g * CH + lp
                for d in range(D // LANES):
                    sl = pl.ds(d * LANES, LANES)
                    partial = []
                    for rref in rrefs:
                        acc = rref[lp * C, sl]
                        for r in range(1, C):
                            acc = acc + rref[lp * C + r, sl]
                        partial.append(acc)
                    outv[pl.ds(pair * D + d * LANES, LANES)] = (
                        partial[0] + partial[1]) + (partial[2] + partial[3])
            return carry

        lax.fori_loop(0, NCHUNK, chunk_body, 0)
        pltpu.sync_copy(outv, out.at[pl.ds(wid * (PPW * D), PPW * D)])

    return body(idx0, idx1, idx2, idx3, t0, t1, t2, t3)


def _dense_all(v3, W_h, b_h, W_g, b_g, W_d, b_d,
               W_diag, b_diag, W_drug, b_drug,
               W_lab, b_lab, W_proc, b_proc):
    vocabs = [W_diag.shape[1], W_drug.shape[1],
              W_lab.shape[1], W_proc.shape[1]]

    def body(v_ref, wh, bh, wg, bg, wd, bd,
             w0, c0, w1, c1, w2, c2, w3, c3,
             o_rd, o_rr, o_rl, o_rp, o_gd, o_gr, o_gl, o_gp,
             o_h, o_vg, o_di, o_gi):
        vb = v_ref[0]
        hb = jnp.tanh(jnp.dot(vb, wh[...],
                              preferred_element_type=jnp.float32) + bh[...])
        vgb = jnp.tanh(jnp.dot(hb, wg[...],
                               preferred_element_type=jnp.float32) + bg[...])
        o_h[0] = hb
        o_vg[0] = vgb
        wdv = wd[...]
        bdv = bd[...]
        o_di[0] = jax.nn.sigmoid(
            jnp.dot(hb, wdv, preferred_element_type=jnp.float32) + bdv)
        o_gi[0] = jax.nn.sigmoid(
            jnp.dot(vgb, wdv, preferred_element_type=jnp.float32) + bdv)
        for wref, cref, o_r, o_g in ((w0, c0, o_rd, o_gd),
                                     (w1, c1, o_rr, o_gr),
                                     (w2, c2, o_rl, o_gl),
                                     (w3, c3, o_rp, o_gp)):
            w = wref[...]
            bb = cref[...]
            o_r[0] = jnp.dot(hb, w, preferred_element_type=jnp.float32) + bb
            o_g[0] = jnp.dot(vgb, w, preferred_element_type=jnp.float32) + bb

    const2 = lambda s: pl.BlockSpec(s, lambda i: (0, 0))
    slab = lambda n: pl.BlockSpec((1, S, n), lambda i: (i, 0, 0))

    out_specs = ([slab(v) for v in vocabs] * 2
                 + [slab(D), slab(D), slab(1), slab(1)])
    out_shape = ([jax.ShapeDtypeStruct((B, S, v), jnp.float32)
                  for v in vocabs] * 2
                 + [jax.ShapeDtypeStruct((B, S, D), jnp.float32),
                    jax.ShapeDtypeStruct((B, S, D), jnp.float32),
                    jax.ShapeDtypeStruct((B, S, 1), jnp.float32),
                    jax.ShapeDtypeStruct((B, S, 1), jnp.float32)])

    return pl.pallas_call(
        body,
        grid=(B,),
        in_specs=[
            pl.BlockSpec((1, S, D), lambda i: (i, 0, 0)),
            const2((D, D)), const2((1, D)),
            const2((D, D)), const2((1, D)),
            const2((D, 1)), const2((1, 1)),
            const2((D, vocabs[0])), const2((1, vocabs[0])),
            const2((D, vocabs[1])), const2((1, vocabs[1])),
            const2((D, vocabs[2])), const2((1, vocabs[2])),
            const2((D, vocabs[3])), const2((1, vocabs[3])),
        ],
        out_specs=out_specs,
        out_shape=out_shape,
    )(v3, W_h, b_h, W_g, b_g, W_d, b_d,
      W_diag, b_diag, W_drug, b_drug, W_lab, b_lab, W_proc, b_proc)


def kernel(diag_seq, drug_seq, lab_seq, proc_seq,
           diag_emb, drug_emb, lab_emb, proc_emb,
           W_h, b_h, W_g, b_g, W_d, b_d,
           W_diag, b_diag, W_drug, b_drug, W_lab, b_lab, W_proc, b_proc):
    idxs = [x.reshape(-1).astype(jnp.int32)
            for x in (diag_seq, drug_seq, lab_seq, proc_seq)]

    v3 = _sc_embed_pool(*idxs, diag_emb, drug_emb, lab_emb,
                        proc_emb).reshape(B, S, D)

    (rdg, rdr, rlb, rpc, gdg, gdr, glb, gpc,
     h3, vg3, rd3, gd3) = _dense_all(
        v3, W_h, b_h.reshape(1, D), W_g, b_g.reshape(1, D),
        W_d, b_d.reshape(1, 1),
        W_diag, b_diag.reshape(1, -1), W_drug, b_drug.reshape(1, -1),
        W_lab, b_lab.reshape(1, -1), W_proc, b_proc.reshape(1, -1))

    return (rdg, rdr, rlb, rpc,
            gdg, gdr, glb, gpc,
            h3, vg3, rd3, gd3)


# speedup vs baseline: 2.2397x; 2.2397x over previous
import functools

import jax
import jax.numpy as jnp
from jax import lax
from jax.experimental import pallas as pl
from jax.experimental.pallas import tpu as pltpu
from jax.experimental.pallas import tpu_sc as plsc

B, S, C, D = 32, 50, 20, 128
PAIRS = B * S
NW = 32
PPW = PAIRS // NW
CH = 2
NCHUNK = PPW // CH
LANES = 16


def _sc_embed_pool(idx0, idx1, idx2, idx3, t0, t1, t2, t3):
    mesh = plsc.VectorSubcoreMesh(core_axis_name="c", subcore_axis_name="s")

    @functools.partial(
        pl.kernel,
        out_type=jax.ShapeDtypeStruct((PAIRS * D,), jnp.float32),
        mesh=mesh,
        scratch_types=[
            pltpu.VMEM((PPW * C,), jnp.int32),
            pltpu.VMEM((PPW * C,), jnp.int32),
            pltpu.VMEM((PPW * C,), jnp.int32),
            pltpu.VMEM((PPW * C,), jnp.int32),
            pltpu.VMEM((CH * C, D), jnp.float32),
            pltpu.VMEM((CH * C, D), jnp.float32),
            pltpu.VMEM((CH * C, D), jnp.float32),
            pltpu.VMEM((CH * C, D), jnp.float32),
            pltpu.VMEM((PPW * D,), jnp.float32),
            pltpu.SemaphoreType.DMA,
        ],
    )
    def body(i0h, i1h, i2h, i3h, e0, e1, e2, e3, out,
             i0, i1, i2, i3, r0, r1, r2, r3, outv, sem):
        wid = lax.axis_index("s") * 2 + lax.axis_index("c")
        ibase = wid * (PPW * C)
        irefs = (i0, i1, i2, i3)
        rrefs = (r0, r1, r2, r3)
        tabs = (e0, e1, e2, e3)
        for iref, ihbm in zip(irefs, (i0h, i1h, i2h, i3h)):
            pltpu.sync_copy(ihbm.at[pl.ds(ibase, PPW * C)], iref)

        def chunk_body(g, carry):
            cps = []
            for iref, tab, rref in zip(irefs, tabs, rrefs):
                cps.append(
                    pltpu.async_copy(
                        tab.at[iref.at[pl.ds(g * (CH * C), CH * C)]], rref, sem
                    )
                )
            for cp in cps:
                cp.wait()
            for lp in range(CH):
                pair = g * CH + lp
                for d in range(D // LANES):
                    sl = pl.ds(d * LANES, LANES)
                    partial = []
                    for rref in rrefs:
                        acc = rref[lp * C, sl]
                        for r in range(1, C):
                            acc = acc + rref[lp * C + r, sl]
                        partial.append(acc)
                    outv[pl.ds(pair * D + d * LANES, LANES)] = (
                        partial[0] + partial[1]) + (partial[2] + partial[3])
            return carry

        lax.fori_loop(0, NCHUNK, chunk_body, 0)
        pltpu.sync_copy(outv, out.at[pl.ds(wid * (PPW * D), PPW * D)])

    return body(idx0, idx1, idx2, idx3, t0, t1, t2, t3)


def _dense_all(vT, W_h, b_h, W_g, b_g, W_d, b_d,
               W_diag, b_diag, W_drug, b_drug,
               W_lab, b_lab, W_proc, b_proc):
    vocabs = [W_diag.shape[1], W_drug.shape[1],
              W_lab.shape[1], W_proc.shape[1]]

    def body(v_ref, wh, bh, wg, bg, wd, bd,
             w0, c0, w1, c1, w2, c2, w3, c3,
             o_rd, o_rr, o_rl, o_rp, o_gd, o_gr, o_gl, o_gp,
             o_h, o_vg, o_di, o_gi):
        vb = v_ref[0]
        hb = jnp.tanh(jnp.dot(vb, wh[...],
                              preferred_element_type=jnp.float32) + bh[...])
        vgb = jnp.tanh(jnp.dot(hb, wg[...],
                               preferred_element_type=jnp.float32) + bg[...])
        o_h[0] = hb
        o_vg[0] = vgb
        wdv = wd[...]
        bdv = bd[...]
        o_di[0] = jax.nn.sigmoid(
            jnp.dot(hb, wdv, preferred_element_type=jnp.float32) + bdv)
        o_gi[0] = jax.nn.sigmoid(
            jnp.dot(vgb, wdv, preferred_element_type=jnp.float32) + bdv)
        for wref, cref, o_r, o_g in ((w0, c0, o_rd, o_gd),
                                     (w1, c1, o_rr, o_gr),
                                     (w2, c2, o_rl, o_gl),
                                     (w3, c3, o_rp, o_gp)):
            w = wref[...]
            bb = cref[...]
            o_r[0] = jnp.dot(hb, w, preferred_element_type=jnp.float32) + bb
            o_g[0] = jnp.dot(vgb, w, preferred_element_type=jnp.float32) + bb

    const2 = lambda s: pl.BlockSpec(s, lambda i: (0, 0))
    slab = lambda n: pl.BlockSpec((1, B, n), lambda i: (i, 0, 0))

    out_specs = ([slab(v) for v in vocabs] * 2
                 + [slab(D), slab(D), slab(1), slab(1)])
    out_shape = ([jax.ShapeDtypeStruct((S, B, v), jnp.float32)
                  for v in vocabs] * 2
                 + [jax.ShapeDtypeStruct((S, B, D), jnp.float32),
                    jax.ShapeDtypeStruct((S, B, D), jnp.float32),
                    jax.ShapeDtypeStruct((S, B, 1), jnp.float32),
                    jax.ShapeDtypeStruct((S, B, 1), jnp.float32)])

    return pl.pallas_call(
        body,
        grid=(S,),
        in_specs=[
            pl.BlockSpec((1, B, D), lambda i: (i, 0, 0)),
            const2((D, D)), const2((1, D)),
            const2((D, D)), const2((1, D)),
            const2((D, 1)), const2((1, 1)),
            const2((D, vocabs[0])), const2((1, vocabs[0])),
            const2((D, vocabs[1])), const2((1, vocabs[1])),
            const2((D, vocabs[2])), const2((1, vocabs[2])),
            const2((D, vocabs[3])), const2((1, vocabs[3])),
        ],
        out_specs=out_specs,
        out_shape=out_shape,
    )(vT, W_h, b_h, W_g, b_g, W_d, b_d,
      W_diag, b_diag, W_drug, b_drug, W_lab, b_lab, W_proc, b_proc)


def kernel(diag_seq, drug_seq, lab_seq, proc_seq,
           diag_emb, drug_emb, lab_emb, proc_emb,
           W_h, b_h, W_g, b_g, W_d, b_d,
           W_diag, b_diag, W_drug, b_drug, W_lab, b_lab, W_proc, b_proc):
    idxs = [x.reshape(-1).astype(jnp.int32)
            for x in (diag_seq, drug_seq, lab_seq, proc_seq)]

    vT = jnp.transpose(
        _sc_embed_pool(*idxs, diag_emb, drug_emb, lab_emb,
                       proc_emb).reshape(B, S, D), (1, 0, 2))

    outs = _dense_all(
        vT, W_h, b_h.reshape(1, D), W_g, b_g.reshape(1, D),
        W_d, b_d.reshape(1, 1),
        W_diag, b_diag.reshape(1, -1), W_drug, b_drug.reshape(1, -1),
        W_lab, b_lab.reshape(1, -1), W_proc, b_proc.reshape(1, -1))

    return tuple(jnp.transpose(o, (1, 0, 2)) for o in outs)
